# baseline (speedup 1.0000x reference)
import functools

import jax
import jax.numpy as jnp
from jax import lax
from jax.experimental import pallas as pl
from jax.experimental.pallas import tpu as pltpu
from jax.experimental.pallas import tpu_sc as plsc

B, L = 4096, 200
V, H = 1000000, 64
C, D = 1000, 32
H2 = 2 * H

NC, NS = 2, 16
NW = NC * NS
BPW = B // NW
CH = 2
NCHUNK = L // CH
NSLOT = 2


def _sc_body(xt_hbm, y_hbm, ftab_hbm, gtab_hbm,
             encx_hbm, sums_hbm, gy_hbm,
             xraw_v, rows_v, outstage_v, acc_v, acct_v, yidx_v, gyrows_v,
             sem_gy, sem_g0, sem_g1, sem_w):
    sem_g = (sem_g0, sem_g1)
    c = lax.axis_index("c")
    s = lax.axis_index("s")
    wid = s * NC + c
    base_b = wid * BPW
    lane = lax.iota(jnp.int32, 16)
    zero = jnp.zeros((16,), jnp.float32)

    pltpu.sync_copy(y_hbm.at[pl.ds(base_b, BPW)], yidx_v)
    pltpu.async_copy(gtab_hbm.at[yidx_v], gyrows_v, sem_gy).wait()
    pltpu.sync_copy(gyrows_v, gy_hbm.at[pl.ds(base_b, BPW)])

    pltpu.sync_copy(xt_hbm.at[:, pl.ds(base_b, BPW)], xraw_v)

    def z_body(h, carry):
        for k in range(8):
            acc_v[h, pl.ds(16 * k, 16)] = zero
        return carry

    lax.fori_loop(0, H, z_body, 0)

    def start_gather(ch, slot):
        for j in range(CH):
            pltpu.async_copy(ftab_hbm.at[xraw_v.at[ch * CH + j]],
                             rows_v.at[slot, pl.ds(j * BPW, BPW)],
                             sem_g[slot])

    def wait_gather(slot):
        for j in range(CH):
            pltpu.make_async_copy(ftab_hbm.at[xraw_v.at[0]],
                                  rows_v.at[slot, pl.ds(j * BPW, BPW)],
                                  sem_g[slot]).wait()

    def shuffle_acc(slot):
        rv = rows_v.at[slot]

        def sh_body(h, carry):
            ti = h >> 3
            r = h & 7
            hvec = jnp.full((16,), 0, jnp.int32) + h
            for j in range(CH):
                for k in range(8):
                    v = plsc.load_gather(rv, [lane + (j * BPW + 16 * k),
                                              hvec])
                    outstage_v[j, ti, r, pl.ds(16 * k, 16)] = v
                    plsc.addupdate(acc_v.at[h, pl.ds(16 * k, 16)], v)
            return carry

        lax.fori_loop(0, H, sh_body, 0)

    def start_write(ch):
        pltpu.async_copy(outstage_v,
                         encx_hbm.at[pl.ds(ch * CH, CH), :, wid], sem_w)

    def wait_write():
        pltpu.make_async_copy(outstage_v,
                              encx_hbm.at[pl.ds(0, CH), :, wid],
                              sem_w).wait()

    start_gather(0, 0)

    def chunk_body(ch, carry):
        for sl in range(NSLOT):
            cc = ch * NSLOT + sl
            wait_gather(sl)

            @pl.when(cc + 1 < NCHUNK)
            def _():
                start_gather(cc + 1, (sl + 1) % NSLOT)

            @pl.when(cc >= 1)
            def _():
                wait_write()

            shuffle_acc(sl)
            start_write(cc)
        return carry

    lax.fori_loop(0, NCHUNK // NSLOT, chunk_body, 0)
    wait_write()

    def at_body(b, carry):
        bvec = jnp.full((16,), 0, jnp.int32) + b
        for k in range(4):
            v = plsc.load_gather(acc_v, [lane + 16 * k, bvec])
            acct_v[b, pl.ds(16 * k, 16)] = v
        return carry

    lax.fori_loop(0, BPW, at_body, 0)
    pltpu.sync_copy(acct_v, sums_hbm.at[pl.ds(base_b, BPW)])


@functools.partial(jax.jit, static_argnums=())
def _sc_gather(xt, y, f_pad, g_table):
    mesh = plsc.VectorSubcoreMesh(core_axis_name="c", subcore_axis_name="s",
                                  num_cores=NC, num_subcores=NS)
    return pl.kernel(
        _sc_body,
        out_type=(
            jax.ShapeDtypeStruct((L, H // 8, B // BPW, 8, BPW),
                                 jnp.float32),
            jax.ShapeDtypeStruct((B, H), jnp.float32),
            jax.ShapeDtypeStruct((B, D), jnp.float32),
        ),
        mesh=mesh,
        scratch_types=[
            pltpu.VMEM((L, BPW), jnp.int32),
            pltpu.VMEM((NSLOT, CH * BPW, H2), jnp.float32),
            pltpu.VMEM((CH, H // 8, 8, BPW), jnp.float32),
            pltpu.VMEM((H, BPW), jnp.float32),
            pltpu.VMEM((BPW, H), jnp.float32),
            pltpu.VMEM((BPW,), jnp.int32),
            pltpu.VMEM((BPW, D), jnp.float32),
            pltpu.SemaphoreType.DMA,
            pltpu.SemaphoreType.DMA,
            pltpu.SemaphoreType.DMA,
            pltpu.SemaphoreType.DMA,
        ],
        compiler_params=pltpu.CompilerParams(use_tc_tiling_on_sc=False,
                                             needs_layout_passes=False),
    )(xt, y, f_pad, g_table)


def _tc_body(sums_ref, gy_ref, y_ref, hW_ref, hb_ref, bW_ref, bb_ref,
             ency_ref, loss_ref):
    i = pl.program_id(0)
    mean_x = sums_ref[...] * (1.0 / L)
    b_x = jnp.tanh(
        lax.dot_general(mean_x, bW_ref[...], (((1,), (1,)), ((), ())),
                        preferred_element_type=jnp.float32) + bb_ref[...])
    enc_y = jnp.tanh(gy_ref[...])
    ae = jnp.tanh(
        lax.dot_general(enc_y, hW_ref[...], (((1,), (1,)), ((), ())),
                        preferred_element_type=jnp.float32) + hb_ref[...])
    m = jnp.max(ae, axis=1, keepdims=True)
    lz = jnp.log(jnp.sum(jnp.exp(ae - m), axis=1, keepdims=True)) + m
    yb = y_ref[...]
    iota_c = lax.broadcasted_iota(jnp.int32, ae.shape, 1)
    pick = jnp.sum(jnp.where(iota_c == yb, ae, 0.0), axis=1, keepdims=True)
    ce_part = jnp.sum(lz - pick)
    mse_part = jnp.sum((b_x - enc_y) ** 2)
    ency_ref[...] = enc_y

    @pl.when(i == 0)
    def _():
        loss_ref[0, 0] = 0.0

    loss_ref[0, 0] += ce_part * (1.0 / B) + mse_part * (1.0 / (B * D))


def _tc_tail(sums, gy, y2d, h_W, h_b2d, b_W, b_b2d):
    blk = 512
    grid = B // blk
    return pl.pallas_call(
        _tc_body,
        grid=(grid,),
        in_specs=[
            pl.BlockSpec((blk, H), lambda i: (i, 0)),
            pl.BlockSpec((blk, D), lambda i: (i, 0)),
            pl.BlockSpec((blk, 1), lambda i: (i, 0)),
            pl.BlockSpec((C, D), lambda i: (0, 0)),
            pl.BlockSpec((1, C), lambda i: (0, 0)),
            pl.BlockSpec((D, H), lambda i: (0, 0)),
            pl.BlockSpec((1, D), lambda i: (0, 0)),
        ],
        out_specs=[
            pl.BlockSpec((blk, D), lambda i: (i, 0)),
            pl.BlockSpec((1, 1), lambda i: (0, 0),
                         memory_space=pltpu.SMEM),
        ],
        out_shape=[
            jax.ShapeDtypeStruct((B, D), jnp.float32),
            jax.ShapeDtypeStruct((1, 1), jnp.float32),
        ],
    )(sums, gy, y2d, h_W, h_b2d, b_W, b_b2d)


def kernel(x, y, f_table, g_table, h_W, h_b, b_W, b_b):
    y32 = y.astype(jnp.int32)
    f_pad = jnp.pad(f_table, ((0, 0), (0, H2 - H)))
    enc5, sums, gy = _sc_gather(x.astype(jnp.int32).T, y32,
                                f_pad, g_table)
    enc_y, loss = _tc_tail(sums, gy, y32.reshape(B, 1), h_W,
                           h_b.reshape(1, C), b_W, b_b.reshape(1, D))
    e = enc5.transpose(2, 4, 0, 1, 3).reshape(B, L, H)
    return (loss[0, 0], e, enc_y)


# speedup vs baseline: 1.4351x; 1.4351x over previous
import functools

import jax
import jax.numpy as jnp
from jax import lax
from jax.experimental import pallas as pl
from jax.experimental.pallas import tpu as pltpu
from jax.experimental.pallas import tpu_sc as plsc

B, L = 4096, 200
V, H = 1000000, 64
C, D = 1000, 32
H2 = 2 * H

NC, NS = 2, 16
NW = NC * NS
BPW = B // NW
CH = 2
NCHUNK = L // CH
NSLOT = 2


def _sc_body(xt_hbm, y_hbm, ftab_hbm, gtab_hbm,
             encx_hbm, sums_hbm, gy_hbm,
             xraw_v, rows_v, outstage_v, acc_v, acct_v, yidx_v, gyrows_v,
             sem_gy, sem_g0, sem_g1, sem_w):
    sem_g = (sem_g0, sem_g1)
    c = lax.axis_index("c")
    s = lax.axis_index("s")
    wid = s * NC + c
    base_b = wid * BPW
    lane = lax.iota(jnp.int32, 16)
    zero = jnp.zeros((16,), jnp.float32)

    pltpu.sync_copy(y_hbm.at[pl.ds(base_b, BPW)], yidx_v)
    pltpu.async_copy(gtab_hbm.at[yidx_v], gyrows_v, sem_gy).wait()
    pltpu.sync_copy(gyrows_v, gy_hbm.at[pl.ds(base_b, BPW)])

    pltpu.sync_copy(xt_hbm.at[:, pl.ds(base_b, BPW)], xraw_v)

    @plsc.parallel_loop(0, H, unroll=2)
    def _(h):
        for k in range(8):
            acc_v[h, pl.ds(16 * k, 16)] = zero

    def start_gather(ch, slot):
        for j in range(CH):
            pltpu.async_copy(ftab_hbm.at[xraw_v.at[ch * CH + j]],
                             rows_v.at[slot, pl.ds(j * BPW, BPW)],
                             sem_g[slot])

    def wait_gather(slot):
        for j in range(CH):
            pltpu.make_async_copy(ftab_hbm.at[xraw_v.at[0]],
                                  rows_v.at[slot, pl.ds(j * BPW, BPW)],
                                  sem_g[slot]).wait()

    def shuffle_acc(slot):
        rv = rows_v.at[slot]

        @plsc.parallel_loop(0, H, unroll=2)
        def _(h):
            ti = h >> 3
            r = h & 7
            hvec = jnp.full((16,), 0, jnp.int32) + h
            for j in range(CH):
                for k in range(8):
                    v = plsc.load_gather(rv, [lane + (j * BPW + 16 * k),
                                              hvec])
                    outstage_v[j, ti, r, pl.ds(16 * k, 16)] = v
                    plsc.addupdate(acc_v.at[h, pl.ds(16 * k, 16)], v)

    def start_write(ch):
        pltpu.async_copy(outstage_v,
                         encx_hbm.at[pl.ds(ch * CH, CH), :, wid], sem_w)

    def wait_write():
        pltpu.make_async_copy(outstage_v,
                              encx_hbm.at[pl.ds(0, CH), :, wid],
                              sem_w).wait()

    start_gather(0, 0)

    def chunk_body(ch, carry):
        for sl in range(NSLOT):
            cc = ch * NSLOT + sl
            wait_gather(sl)

            @pl.when(cc + 1 < NCHUNK)
            def _():
                start_gather(cc + 1, (sl + 1) % NSLOT)

            @pl.when(cc >= 1)
            def _():
                wait_write()

            shuffle_acc(sl)
            start_write(cc)
        return carry

    lax.fori_loop(0, NCHUNK // NSLOT, chunk_body, 0)
    wait_write()

    @plsc.parallel_loop(0, BPW, unroll=2)
    def _(b):
        bvec = jnp.full((16,), 0, jnp.int32) + b
        for k in range(4):
            v = plsc.load_gather(acc_v, [lane + 16 * k, bvec])
            acct_v[b, pl.ds(16 * k, 16)] = v
    pltpu.sync_copy(acct_v, sums_hbm.at[pl.ds(base_b, BPW)])


@functools.partial(jax.jit, static_argnums=())
def _sc_gather(xt, y, f_pad, g_table):
    mesh = plsc.VectorSubcoreMesh(core_axis_name="c", subcore_axis_name="s",
                                  num_cores=NC, num_subcores=NS)
    return pl.kernel(
        _sc_body,
        out_type=(
            jax.ShapeDtypeStruct((L, H // 8, B // BPW, 8, BPW),
                                 jnp.float32),
            jax.ShapeDtypeStruct((B, H), jnp.float32),
            jax.ShapeDtypeStruct((B, D), jnp.float32),
        ),
        mesh=mesh,
        scratch_types=[
            pltpu.VMEM((L, BPW), jnp.int32),
            pltpu.VMEM((NSLOT, CH * BPW, H2), jnp.float32),
            pltpu.VMEM((CH, H // 8, 8, BPW), jnp.float32),
            pltpu.VMEM((H, BPW), jnp.float32),
            pltpu.VMEM((BPW, H), jnp.float32),
            pltpu.VMEM((BPW,), jnp.int32),
            pltpu.VMEM((BPW, D), jnp.float32),
            pltpu.SemaphoreType.DMA,
            pltpu.SemaphoreType.DMA,
            pltpu.SemaphoreType.DMA,
            pltpu.SemaphoreType.DMA,
        ],
        compiler_params=pltpu.CompilerParams(use_tc_tiling_on_sc=False,
                                             needs_layout_passes=False),
    )(xt, y, f_pad, g_table)


def _tc_body(sums_ref, gy_ref, y_ref, hW_ref, hb_ref, bW_ref, bb_ref,
             ency_ref, loss_ref):
    i = pl.program_id(0)
    mean_x = sums_ref[...] * (1.0 / L)
    b_x = jnp.tanh(
        lax.dot_general(mean_x, bW_ref[...], (((1,), (1,)), ((), ())),
                        preferred_element_type=jnp.float32) + bb_ref[...])
    enc_y = jnp.tanh(gy_ref[...])
    ae = jnp.tanh(
        lax.dot_general(enc_y, hW_ref[...], (((1,), (1,)), ((), ())),
                        preferred_element_type=jnp.float32) + hb_ref[...])
    m = jnp.max(ae, axis=1, keepdims=True)
    lz = jnp.log(jnp.sum(jnp.exp(ae - m), axis=1, keepdims=True)) + m
    yb = y_ref[...]
    iota_c = lax.broadcasted_iota(jnp.int32, ae.shape, 1)
    pick = jnp.sum(jnp.where(iota_c == yb, ae, 0.0), axis=1, keepdims=True)
    ce_part = jnp.sum(lz - pick)
    mse_part = jnp.sum((b_x - enc_y) ** 2)
    ency_ref[...] = enc_y

    @pl.when(i == 0)
    def _():
        loss_ref[0, 0] = 0.0

    loss_ref[0, 0] += ce_part * (1.0 / B) + mse_part * (1.0 / (B * D))


def _tc_tail(sums, gy, y2d, h_W, h_b2d, b_W, b_b2d):
    blk = 512
    grid = B // blk
    return pl.pallas_call(
        _tc_body,
        grid=(grid,),
        in_specs=[
            pl.BlockSpec((blk, H), lambda i: (i, 0)),
            pl.BlockSpec((blk, D), lambda i: (i, 0)),
            pl.BlockSpec((blk, 1), lambda i: (i, 0)),
            pl.BlockSpec((C, D), lambda i: (0, 0)),
            pl.BlockSpec((1, C), lambda i: (0, 0)),
            pl.BlockSpec((D, H), lambda i: (0, 0)),
            pl.BlockSpec((1, D), lambda i: (0, 0)),
        ],
        out_specs=[
            pl.BlockSpec((blk, D), lambda i: (i, 0)),
            pl.BlockSpec((1, 1), lambda i: (0, 0),
                         memory_space=pltpu.SMEM),
        ],
        out_shape=[
            jax.ShapeDtypeStruct((B, D), jnp.float32),
            jax.ShapeDtypeStruct((1, 1), jnp.float32),
        ],
    )(sums, gy, y2d, h_W, h_b2d, b_W, b_b2d)


def kernel(x, y, f_table, g_table, h_W, h_b, b_W, b_b):
    y32 = y.astype(jnp.int32)
    f_pad = jnp.pad(f_table, ((0, 0), (0, H2 - H)))
    enc5, sums, gy = _sc_gather(x.astype(jnp.int32).T, y32,
                                f_pad, g_table)
    enc_y, loss = _tc_tail(sums, gy, y32.reshape(B, 1), h_W,
                           h_b.reshape(1, C), b_W, b_b.reshape(1, D))
    e = enc5.transpose(2, 4, 0, 1, 3).reshape(B, L, H)
    return (loss[0, 0], e, enc_y)


# speedup vs baseline: 1.8480x; 1.2878x over previous
import functools

import jax
import jax.numpy as jnp
from jax import lax
from jax.experimental import pallas as pl
from jax.experimental.pallas import tpu as pltpu
from jax.experimental.pallas import tpu_sc as plsc

B, L = 4096, 200
V, H = 1000000, 64
C, D = 1000, 32
H2 = 2 * H

NC, NS = 2, 16
NW = NC * NS
BPW = B // NW
SPLIT = 104
NSLOT = 3


def _sc_body(xt_hbm, y_hbm, ftab_hbm, gtab_hbm,
             encx_hbm, sums_hbm, gy_hbm,
             xraw_v, idx_v, rows_v, acc_v, yidx_v, gyrows_v,
             sem_gy, *sems):
    sem_g = sems[:NSLOT]
    sem_w = sems[NSLOT:]
    c = lax.axis_index("c")
    s = lax.axis_index("s")
    wid = s * NC + c
    base_b = wid * BPW

    pltpu.sync_copy(y_hbm.at[pl.ds(base_b, BPW)], yidx_v)
    pltpu.async_copy(gtab_hbm.at[yidx_v], gyrows_v, sem_gy).wait()
    pltpu.sync_copy(gyrows_v, gy_hbm.at[pl.ds(base_b, BPW)])

    lane = lax.iota(jnp.int32, 16)

    def stage_positions(l_lo, n_l, blocks):
        pltpu.sync_copy(xt_hbm.at[pl.ds(l_lo, n_l), pl.ds(base_b, BPW)],
                        xraw_v.at[pl.ds(0, n_l), :])

        def tr_body(b, carry):
            bvec = jnp.full((16,), 0, jnp.int32) + b
            for l0 in blocks:
                v = plsc.load_gather(xraw_v, [lane + l0, bvec])
                idx_v[b, pl.ds(l_lo + l0, 16)] = v
            return carry

        lax.fori_loop(0, BPW, tr_body, 0)

    stage_positions(0, SPLIT, [0, 16, 32, 48, 64, 80, 88])
    stage_positions(SPLIT, L - SPLIT, [0, 16, 32, 48, 64, 80])

    def start_gather(b, slot):
        pltpu.async_copy(ftab_hbm.at[idx_v.at[b, pl.ds(0, SPLIT)]],
                         rows_v.at[slot, pl.ds(0, SPLIT)], sem_g[slot])
        pltpu.async_copy(ftab_hbm.at[idx_v.at[b, pl.ds(SPLIT, L - SPLIT)]],
                         rows_v.at[slot, pl.ds(SPLIT, L - SPLIT)], sem_g[slot])

    def wait_gather(slot):
        pltpu.make_async_copy(ftab_hbm.at[idx_v.at[0, pl.ds(0, SPLIT)]],
                              rows_v.at[slot, pl.ds(0, SPLIT)],
                              sem_g[slot]).wait()
        pltpu.make_async_copy(ftab_hbm.at[idx_v.at[0, pl.ds(SPLIT, L - SPLIT)]],
                              rows_v.at[slot, pl.ds(SPLIT, L - SPLIT)],
                              sem_g[slot]).wait()

    def start_write(b, slot):
        pltpu.async_copy(rows_v.at[slot], encx_hbm.at[base_b + b], sem_w[slot])

    def wait_write(slot):
        pltpu.make_async_copy(rows_v.at[slot], encx_hbm.at[0],
                              sem_w[slot]).wait()

    def accumulate(b, slot):
        def acc_body(r, a):
            a0, a1, a2, a3 = a
            return (a0 + rows_v[slot, r, 0:16],
                    a1 + rows_v[slot, r, 16:32],
                    a2 + rows_v[slot, r, 32:48],
                    a3 + rows_v[slot, r, 48:64])

        z = jnp.zeros((16,), jnp.float32)
        a0, a1, a2, a3 = lax.fori_loop(0, L, acc_body, (z, z, z, z),
                                       unroll=4)
        acc_v[b, 0:16] = a0
        acc_v[b, 16:32] = a1
        acc_v[b, 32:48] = a2
        acc_v[b, 48:64] = a3

    start_gather(0, 0)
    start_gather(1, 1)

    def do_row(b, sl):
        wait_gather(sl)
        start_write(b, sl)
        nxt = (sl + 2) % NSLOT

        @pl.when(b >= 1)
        def _():
            wait_write(nxt)

        @pl.when(b + 2 < BPW)
        def _():
            start_gather(b + 2, nxt)

        accumulate(b, sl)

    def group_body(g, carry):
        for sl in range(NSLOT):
            do_row(g * NSLOT + sl, sl)
        return carry

    lax.fori_loop(0, BPW // NSLOT, group_body, 0)
    do_row(126, 0)
    do_row(127, 1)
    wait_write(1)
    pltpu.sync_copy(acc_v, sums_hbm.at[pl.ds(base_b, BPW)])


@functools.partial(jax.jit, static_argnums=())
def _sc_gather(xt, y, f_pad, g_table):
    mesh = plsc.VectorSubcoreMesh(core_axis_name="c", subcore_axis_name="s",
                                  num_cores=NC, num_subcores=NS)
    return pl.kernel(
        _sc_body,
        out_type=(
            jax.ShapeDtypeStruct((B, L, H2), jnp.float32),
            jax.ShapeDtypeStruct((B, H), jnp.float32),
            jax.ShapeDtypeStruct((B, D), jnp.float32),
        ),
        mesh=mesh,
        scratch_types=[
            pltpu.VMEM((SPLIT, BPW), jnp.int32),
            pltpu.VMEM((BPW, L), jnp.int32),
            pltpu.VMEM((NSLOT, L, H2), jnp.float32),
            pltpu.VMEM((BPW, H), jnp.float32),
            pltpu.VMEM((BPW,), jnp.int32),
            pltpu.VMEM((BPW, D), jnp.float32),
        ] + [pltpu.SemaphoreType.DMA] * (1 + 2 * NSLOT),
        compiler_params=pltpu.CompilerParams(use_tc_tiling_on_sc=False,
                                             needs_layout_passes=False),
    )(xt, y, f_pad, g_table)


def _tc_body(sums_ref, gy_ref, y_ref, hW_ref, hb_ref, bW_ref, bb_ref,
             ency_ref, loss_ref):
    i = pl.program_id(0)
    mean_x = sums_ref[...] * (1.0 / L)
    b_x = jnp.tanh(
        lax.dot_general(mean_x, bW_ref[...], (((1,), (1,)), ((), ())),
                        preferred_element_type=jnp.float32) + bb_ref[...])
    enc_y = jnp.tanh(gy_ref[...])
    ae = jnp.tanh(
        lax.dot_general(enc_y, hW_ref[...], (((1,), (1,)), ((), ())),
                        preferred_element_type=jnp.float32) + hb_ref[...])
    m = jnp.max(ae, axis=1, keepdims=True)
    lz = jnp.log(jnp.sum(jnp.exp(ae - m), axis=1, keepdims=True)) + m
    yb = y_ref[...]
    iota_c = lax.broadcasted_iota(jnp.int32, ae.shape, 1)
    pick = jnp.sum(jnp.where(iota_c == yb, ae, 0.0), axis=1, keepdims=True)
    ce_part = jnp.sum(lz - pick)
    mse_part = jnp.sum((b_x - enc_y) ** 2)
    ency_ref[...] = enc_y

    @pl.when(i == 0)
    def _():
        loss_ref[0, 0] = 0.0

    loss_ref[0, 0] += ce_part * (1.0 / B) + mse_part * (1.0 / (B * D))


def _tc_tail(sums, gy, y2d, h_W, h_b2d, b_W, b_b2d):
    blk = 512
    grid = B // blk
    return pl.pallas_call(
        _tc_body,
        grid=(grid,),
        in_specs=[
            pl.BlockSpec((blk, H), lambda i: (i, 0)),
            pl.BlockSpec((blk, D), lambda i: (i, 0)),
            pl.BlockSpec((blk, 1), lambda i: (i, 0)),
            pl.BlockSpec((C, D), lambda i: (0, 0)),
            pl.BlockSpec((1, C), lambda i: (0, 0)),
            pl.BlockSpec((D, H), lambda i: (0, 0)),
            pl.BlockSpec((1, D), lambda i: (0, 0)),
        ],
        out_specs=[
            pl.BlockSpec((blk, D), lambda i: (i, 0)),
            pl.BlockSpec((1, 1), lambda i: (0, 0),
                         memory_space=pltpu.SMEM),
        ],
        out_shape=[
            jax.ShapeDtypeStruct((B, D), jnp.float32),
            jax.ShapeDtypeStruct((1, 1), jnp.float32),
        ],
    )(sums, gy, y2d, h_W, h_b2d, b_W, b_b2d)


def kernel(x, y, f_table, g_table, h_W, h_b, b_W, b_b):
    y32 = y.astype(jnp.int32)
    f_pad = jnp.pad(f_table, ((0, 0), (0, H2 - H)))
    enc_xp, sums, gy = _sc_gather(x.astype(jnp.int32).T, y32,
                                  f_pad, g_table)
    enc_y, loss = _tc_tail(sums, gy, y32.reshape(B, 1), h_W,
                           h_b.reshape(1, C), b_W, b_b.reshape(1, D))
    return (loss[0, 0], enc_xp[:, :, :H], enc_y)
